# Initial kernel scaffold; baseline (speedup 1.0000x reference)
import jax
import jax.numpy as jnp
from jax.experimental import pallas as pl


def kernel(x, edge_index, W1, b1, Wmu, bmu, Wls, bls):
    raise NotImplementedError("write your pallas kernel here")


# speedup vs baseline: 68.4134x; 68.4134x over previous
import functools

import jax
import jax.numpy as jnp
from jax import lax
from jax.experimental import pallas as pl
from jax.experimental.pallas import tpu as pltpu
from jax.experimental.pallas import tpu_sc as plsc

_N = 10000
_E = 320000
_D_IN = 128
_D_HID = 4
_D_OUT = 2

_NC = 2
_NS = 16
_NW = _NC * _NS

_CH = 128
_NCHUNK = 79
_E_PER = _NCHUNK * _CH
_E_PAD = _NW * _E_PER

_ROWS_SUB = 640
_NP = _NS * _ROWS_SUB


_sc_mesh = plsc.VectorSubcoreMesh(core_axis_name="c", subcore_axis_name="s")


@functools.partial(
    pl.kernel,
    out_type=jax.ShapeDtypeStruct((_NC, _NP), jnp.float32),
    mesh=_sc_mesh,
    scratch_types=[
        pltpu.VMEM((_NCHUNK, _CH), jnp.int32),
        pltpu.VMEM((_CH,), jnp.float32),
        pltpu.VMEM((_ROWS_SUB,), jnp.float32),
        pltpu.VMEM_SHARED((_NP,), jnp.float32),
    ],
)
def _deg_kernel(dst_hbm, out_hbm, dst_v, ones_v, bounce_v, acc_sh):
    cid = lax.axis_index("c")
    sid = lax.axis_index("s")
    w = sid * _NC + cid
    nbase = sid * _ROWS_SUB
    pltpu.sync_copy(dst_hbm.at[w], dst_v)

    def fill16(i, buf, val):
        buf[pl.ds(i * 16, 16)] = jnp.full((16,), val, jnp.float32)

    lax.fori_loop(0, _CH // 16, lambda i, _: (fill16(i, ones_v, 1.0), None)[1], None)
    lax.fori_loop(0, _ROWS_SUB // 16, lambda i, _: (fill16(i, bounce_v, 0.0), None)[1], None)
    pltpu.sync_copy(bounce_v, acc_sh.at[pl.ds(nbase, _ROWS_SUB)])
    plsc.subcore_barrier()

    def body(j, _):
        pltpu.sync_copy(ones_v, acc_sh.at[dst_v.at[j]], add=True)
        return None

    lax.fori_loop(0, _NCHUNK, body, None)
    plsc.subcore_barrier()
    pltpu.sync_copy(acc_sh.at[pl.ds(nbase, _ROWS_SUB)], bounce_v)
    pltpu.sync_copy(bounce_v, out_hbm.at[cid, pl.ds(nbase, _ROWS_SUB)])


@functools.partial(
    pl.kernel,
    out_type=jax.ShapeDtypeStruct((_NC, _D_HID, _NP), jnp.float32),
    mesh=_sc_mesh,
    scratch_types=[
        pltpu.VMEM((_NCHUNK, _CH), jnp.int32),
        pltpu.VMEM((_NCHUNK, _CH), jnp.int32),
        pltpu.VMEM((_D_HID * _NP,), jnp.float32),
        [pltpu.VMEM((_CH,), jnp.float32) for _ in range(_D_HID)],
        [pltpu.VMEM((_CH,), jnp.int32) for _ in range(_D_HID)],
        pltpu.VMEM((_ROWS_SUB,), jnp.float32),
        pltpu.VMEM_SHARED((_D_HID * _NP,), jnp.float32),
    ],
    compiler_params=pltpu.CompilerParams(needs_layout_passes=False),
)
def _agg_kernel(src_hbm, dst_hbm, g_hbm, out_hbm,
                src_v, dst_v, g_v, vals_vs, ixo_vs, bounce_v, acc_sh):
    cid = lax.axis_index("c")
    sid = lax.axis_index("s")
    w = sid * _NC + cid
    nbase = sid * _ROWS_SUB
    pltpu.sync_copy(src_hbm.at[w], src_v)
    pltpu.sync_copy(dst_hbm.at[w], dst_v)
    for c in range(_D_HID):
        pltpu.sync_copy(g_hbm.at[c], g_v.at[pl.ds(c * _NP, _NP)])

    def fill16(i, buf):
        buf[pl.ds(i * 16, 16)] = jnp.zeros((16,), jnp.float32)

    lax.fori_loop(0, _ROWS_SUB // 16, lambda i, _: (fill16(i, bounce_v), None)[1], None)
    for c in range(_D_HID):
        pltpu.sync_copy(bounce_v, acc_sh.at[pl.ds(c * _NP + nbase, _ROWS_SUB)])
    plsc.subcore_barrier()

    def body(j, _):
        for k in range(_CH // 16):
            sl = pl.ds(16 * k, 16)
            idx = src_v[j, sl]
            didx = dst_v[j, sl]
            for c in range(_D_HID):
                vals_vs[c][sl] = plsc.load_gather(g_v, [idx + c * _NP])
                ixo_vs[c][sl] = didx + c * _NP
        for c in range(_D_HID):
            pltpu.sync_copy(vals_vs[c], acc_sh.at[ixo_vs[c]], add=True)
        return None

    lax.fori_loop(0, _NCHUNK, body, None)
    plsc.subcore_barrier()
    for c in range(_D_HID):
        pltpu.sync_copy(acc_sh.at[pl.ds(c * _NP + nbase, _ROWS_SUB)], bounce_v)
        pltpu.sync_copy(bounce_v, out_hbm.at[cid, c, pl.ds(nbase, _ROWS_SUB)])


def _tc1_body(x_ref, w1_ref, deg_ref, dinv_ref, g1_ref):
    deg = deg_ref[0:1, :] + deg_ref[1:2, :] + 1.0
    dinv = lax.rsqrt(deg)
    dinv_ref[...] = dinv
    h0 = lax.dot_general(w1_ref[...], x_ref[...], (((0,), (1,)), ((), ())),
                         preferred_element_type=jnp.float32)
    g1_ref[...] = h0 * dinv


_tc1 = pl.pallas_call(
    _tc1_body,
    out_shape=[
        jax.ShapeDtypeStruct((1, _NP), jnp.float32),
        jax.ShapeDtypeStruct((_D_HID, _NP), jnp.float32),
    ],
)


def _tc2_body(t1_ref, g1_ref, dinv_ref, b1_ref, g2_ref):
    dinv = dinv_ref[...]
    t1 = t1_ref[0] + t1_ref[1] + g1_ref[...]
    h = jnp.maximum(dinv * t1 + b1_ref[...], 0.0)
    g2_ref[...] = dinv * h


_tc2 = pl.pallas_call(
    _tc2_body,
    out_shape=jax.ShapeDtypeStruct((_D_HID, _NP), jnp.float32),
)


def _tc3_body(t2_ref, g2_ref, dinv_ref, wmu_ref, bmu_ref,
              wls_ref, bls_ref, mu_ref, ls_ref):
    hag = dinv_ref[...] * (t2_ref[0] + t2_ref[1] + g2_ref[...])
    mu_ref[...] = lax.dot_general(wmu_ref[...], hag, (((0,), (0,)), ((), ())),
                                  preferred_element_type=jnp.float32) + bmu_ref[...]
    ls_ref[...] = lax.dot_general(wls_ref[...], hag, (((0,), (0,)), ((), ())),
                                  preferred_element_type=jnp.float32) + bls_ref[...]


_tc3 = pl.pallas_call(
    _tc3_body,
    out_shape=[
        jax.ShapeDtypeStruct((_D_OUT, _NP), jnp.float32),
        jax.ShapeDtypeStruct((_D_OUT, _NP), jnp.float32),
    ],
)


def kernel(x, edge_index, W1, b1, Wmu, bmu, Wls, bls):
    src = edge_index[0]
    dst = edge_index[1]
    pad = _N + (jnp.arange(_E_PAD - _E, dtype=jnp.int32) % (_NP - _N))
    src3 = jnp.concatenate([src, pad]).reshape(_NW, _NCHUNK, _CH)
    dst3 = jnp.concatenate([dst, pad]).reshape(_NW, _NCHUNK, _CH)
    x_pad = jnp.zeros((_NP, _D_IN), jnp.float32).at[:_N].set(x)

    deg = _deg_kernel(dst3)
    dinv, g1 = _tc1(x_pad, W1, deg)
    t1 = _agg_kernel(src3, dst3, g1)
    g2 = _tc2(t1, g1, dinv, b1.reshape(_D_HID, 1))
    t2 = _agg_kernel(src3, dst3, g2)
    mu, ls = _tc3(t2, g2, dinv, Wmu, bmu.reshape(_D_OUT, 1),
                  Wls, bls.reshape(_D_OUT, 1))
    return mu[:, :_N].T, ls[:, :_N].T


# speedup vs baseline: 100.4293x; 1.4680x over previous
import functools

import jax
import jax.numpy as jnp
from jax import lax
from jax.experimental import pallas as pl
from jax.experimental.pallas import tpu as pltpu
from jax.experimental.pallas import tpu_sc as plsc

_N = 10000
_E = 320000
_D_IN = 128
_D_HID = 4
_D_OUT = 2

_NC = 2
_NS = 16
_NW = _NC * _NS

_CH = 128
_NCHUNK = 79
_E_PER = _NCHUNK * _CH
_E_PAD = _NW * _E_PER

_ROWS_SUB = 640
_NP = _NS * _ROWS_SUB


_sc_mesh = plsc.VectorSubcoreMesh(core_axis_name="c", subcore_axis_name="s")


@functools.partial(
    pl.kernel,
    out_type=jax.ShapeDtypeStruct((_NC, _NP), jnp.float32),
    mesh=_sc_mesh,
    scratch_types=[
        pltpu.VMEM((_NCHUNK, _CH), jnp.int32),
        pltpu.VMEM((_CH,), jnp.float32),
        pltpu.VMEM((_ROWS_SUB,), jnp.float32),
        pltpu.VMEM_SHARED((_NP,), jnp.float32),
    ],
)
def _deg_kernel(dst_hbm, out_hbm, dst_v, ones_v, bounce_v, acc_sh):
    cid = lax.axis_index("c")
    sid = lax.axis_index("s")
    w = sid * _NC + cid
    nbase = sid * _ROWS_SUB
    pltpu.sync_copy(dst_hbm.at[w], dst_v)

    def fill16(i, buf, val):
        buf[pl.ds(i * 16, 16)] = jnp.full((16,), val, jnp.float32)

    lax.fori_loop(0, _CH // 16, lambda i, _: (fill16(i, ones_v, 1.0), None)[1], None)
    lax.fori_loop(0, _ROWS_SUB // 16, lambda i, _: (fill16(i, bounce_v, 0.0), None)[1], None)
    pltpu.sync_copy(bounce_v, acc_sh.at[pl.ds(nbase, _ROWS_SUB)])
    plsc.subcore_barrier()

    def body(j, _):
        pltpu.sync_copy(ones_v, acc_sh.at[dst_v.at[j]], add=True)
        return None

    lax.fori_loop(0, _NCHUNK, body, None)
    plsc.subcore_barrier()
    pltpu.sync_copy(acc_sh.at[pl.ds(nbase, _ROWS_SUB)], bounce_v)
    pltpu.sync_copy(bounce_v, out_hbm.at[cid, pl.ds(nbase, _ROWS_SUB)])


@functools.partial(
    pl.kernel,
    out_type=jax.ShapeDtypeStruct((_NC, _D_HID, _NP), jnp.float32),
    mesh=_sc_mesh,
    scratch_types=[
        pltpu.VMEM((_NCHUNK, _CH), jnp.int32),
        pltpu.VMEM((_NCHUNK, _CH), jnp.int32),
        pltpu.VMEM((_D_HID * _NP,), jnp.float32),
        [pltpu.VMEM((_D_HID * _CH,), jnp.float32) for _ in range(2)],
        [pltpu.VMEM((_D_HID * _CH,), jnp.int32) for _ in range(2)],
        pltpu.VMEM((_ROWS_SUB,), jnp.float32),
        pltpu.VMEM_SHARED((_D_HID * _NP,), jnp.float32),
        [pltpu.SemaphoreType.DMA for _ in range(2)],
    ],
    compiler_params=pltpu.CompilerParams(needs_layout_passes=False),
)
def _agg_kernel(src_hbm, dst_hbm, g_hbm, out_hbm,
                src_v, dst_v, g_v, vals_vs, ixo_vs, bounce_v, acc_sh, sems):
    cid = lax.axis_index("c")
    sid = lax.axis_index("s")
    w = sid * _NC + cid
    nbase = sid * _ROWS_SUB
    pltpu.sync_copy(src_hbm.at[w], src_v)
    pltpu.sync_copy(dst_hbm.at[w], dst_v)
    for c in range(_D_HID):
        pltpu.sync_copy(g_hbm.at[c], g_v.at[pl.ds(c * _NP, _NP)])

    def fill16(i, buf):
        buf[pl.ds(i * 16, 16)] = jnp.zeros((16,), jnp.float32)

    lax.fori_loop(0, _ROWS_SUB // 16, lambda i, _: (fill16(i, bounce_v), None)[1], None)
    for c in range(_D_HID):
        pltpu.sync_copy(bounce_v, acc_sh.at[pl.ds(c * _NP + nbase, _ROWS_SUB)])
    plsc.subcore_barrier()

    def compute(j, b):
        for k in range(_CH // 16):
            sl = pl.ds(16 * k, 16)
            idx = src_v[j, sl]
            didx = dst_v[j, sl]
            for c in range(_D_HID):
                off = c * _CH + 16 * k
                vals_vs[b][pl.ds(off, 16)] = plsc.load_gather(g_v, [idx + c * _NP])
                ixo_vs[b][pl.ds(off, 16)] = didx + c * _NP

    def issue(b):
        pltpu.async_copy(vals_vs[b], acc_sh.at[ixo_vs[b]], sems[b], add=True)

    def drain(b):
        pltpu.make_async_copy(vals_vs[b], acc_sh.at[ixo_vs[b]], sems[b]).wait()

    compute(0, 0)
    issue(0)

    def body(i, _):
        compute(2 * i + 1, 1)
        drain(0)
        issue(1)
        compute(2 * i + 2, 0)
        drain(1)
        issue(0)
        return None

    lax.fori_loop(0, (_NCHUNK - 1) // 2, body, None)
    drain(0)
    plsc.subcore_barrier()
    for c in range(_D_HID):
        pltpu.sync_copy(acc_sh.at[pl.ds(c * _NP + nbase, _ROWS_SUB)], bounce_v)
        pltpu.sync_copy(bounce_v, out_hbm.at[cid, c, pl.ds(nbase, _ROWS_SUB)])


def _tc1_body(x_ref, w1_ref, deg_ref, dinv_ref, g1_ref):
    deg = deg_ref[0:1, :] + deg_ref[1:2, :] + 1.0
    r = lax.rsqrt(deg)
    dinv = r * (1.5 - 0.5 * deg * r * r)
    dinv_ref[...] = dinv
    h0 = lax.dot_general(w1_ref[...], x_ref[...], (((0,), (1,)), ((), ())),
                         preferred_element_type=jnp.float32,
                         precision=lax.Precision.HIGHEST)
    g1_ref[...] = h0 * dinv


_tc1 = pl.pallas_call(
    _tc1_body,
    out_shape=[
        jax.ShapeDtypeStruct((1, _NP), jnp.float32),
        jax.ShapeDtypeStruct((_D_HID, _NP), jnp.float32),
    ],
)


def _tc2_body(t1_ref, g1_ref, dinv_ref, b1_ref, g2_ref):
    dinv = dinv_ref[...]
    t1 = t1_ref[0] + t1_ref[1] + g1_ref[...]
    h = jnp.maximum(dinv * t1 + b1_ref[...], 0.0)
    g2_ref[...] = dinv * h


_tc2 = pl.pallas_call(
    _tc2_body,
    out_shape=jax.ShapeDtypeStruct((_D_HID, _NP), jnp.float32),
)


def _tc3_body(t2_ref, g2_ref, dinv_ref, wmu_ref, bmu_ref,
              wls_ref, bls_ref, mu_ref, ls_ref):
    hag = dinv_ref[...] * (t2_ref[0] + t2_ref[1] + g2_ref[...])
    mu_ref[...] = lax.dot_general(wmu_ref[...], hag, (((0,), (0,)), ((), ())),
                                  preferred_element_type=jnp.float32,
                                  precision=lax.Precision.HIGHEST) + bmu_ref[...]
    ls_ref[...] = lax.dot_general(wls_ref[...], hag, (((0,), (0,)), ((), ())),
                                  preferred_element_type=jnp.float32,
                                  precision=lax.Precision.HIGHEST) + bls_ref[...]


_tc3 = pl.pallas_call(
    _tc3_body,
    out_shape=[
        jax.ShapeDtypeStruct((_D_OUT, _NP), jnp.float32),
        jax.ShapeDtypeStruct((_D_OUT, _NP), jnp.float32),
    ],
)


def kernel(x, edge_index, W1, b1, Wmu, bmu, Wls, bls):
    src = edge_index[0]
    dst = edge_index[1]
    pad = _N + (jnp.arange(_E_PAD - _E, dtype=jnp.int32) % (_NP - _N))
    src3 = jnp.concatenate([src, pad]).reshape(_NW, _NCHUNK, _CH)
    dst3 = jnp.concatenate([dst, pad]).reshape(_NW, _NCHUNK, _CH)
    x_pad = jnp.zeros((_NP, _D_IN), jnp.float32).at[:_N].set(x)

    deg = _deg_kernel(dst3)
    dinv, g1 = _tc1(x_pad, W1, deg)
    t1 = _agg_kernel(src3, dst3, g1)
    g2 = _tc2(t1, g1, dinv, b1.reshape(_D_HID, 1))
    t2 = _agg_kernel(src3, dst3, g2)
    mu, ls = _tc3(t2, g2, dinv, Wmu, bmu.reshape(_D_OUT, 1),
                  Wls, bls.reshape(_D_OUT, 1))
    return mu[:, :_N].T, ls[:, :_N].T
